# baseline (speedup 1.0000x reference)
import jax
import jax.numpy as jnp
from jax import lax
from jax.experimental import pallas as pl
from jax.experimental.pallas import tpu as pltpu


def _pos_body(row_ref, col_ref, out_ref, pos_scratch):
    i = pl.program_id(0)

    @pl.when(i == 0)
    def _():
        p = lax.broadcasted_iota(jnp.int32, (32, 1024), 1)
        g = lax.broadcasted_iota(jnp.int32, (32, 1024), 0)
        sel_w = (p % 32 == g).astype(jnp.float32)
        sel_h = (p // 32 == g).astype(jnp.float32)
        dn = (((0,), (0,)), ((), ()))
        top = lax.dot_general(col_ref[0:32, :], sel_w, dn,
                              preferred_element_type=jnp.float32)
        bot = lax.dot_general(row_ref[0:32, :], sel_h, dn,
                              preferred_element_type=jnp.float32)
        pos_scratch[0:256, :] = top
        pos_scratch[256:512, :] = bot

    out_ref[0, :, :] = pos_scratch[:, :]


def kernel(x, row_embed, col_embed):
    b = x.shape[0]
    out = pl.pallas_call(
        _pos_body,
        grid=(b,),
        in_specs=[
            pl.BlockSpec((50, 256), lambda i: (0, 0)),
            pl.BlockSpec((50, 256), lambda i: (0, 0)),
        ],
        out_specs=pl.BlockSpec((1, 512, 1024), lambda i: (i, 0, 0)),
        out_shape=jax.ShapeDtypeStruct((b, 512, 1024), jnp.float32),
        scratch_shapes=[pltpu.VMEM((512, 1024), jnp.float32)],
    )(row_embed, col_embed)
    return out.reshape(b, 512, 32, 32)
